# Initial kernel scaffold; baseline (speedup 1.0000x reference)
import jax
import jax.numpy as jnp
from jax.experimental import pallas as pl


def kernel(block_outputs, pos_edge_index, neg_edge_index):
    raise NotImplementedError("write your pallas kernel here")


# speedup vs baseline: 1.2552x; 1.2552x over previous
import functools

import jax
import jax.numpy as jnp
import numpy as np
from jax import lax
from jax.experimental import pallas as pl
from jax.experimental.pallas import tpu as pltpu
from jax.experimental.pallas import tpu_sc as plsc

_N_NODES = 10000
_D = 128
_N_POS = 320000
_N_NEG = 320000
_E = _N_POS + _N_NEG
_PROB = _N_POS / (_N_NODES**2 - _N_NODES) * 2
_EPS = float(-np.log(1.0 - _PROB))

_NC = 2
_NS = 16
_NW = _NC * _NS
_EPW = _E // _NW
_B = 80
_NBLK = _EPW // _B
_LANES = 16


def _sc_scores_kernel(h_hbm, u_hbm, v_hbm, out_hbm,
                      iu, iv, ru0, rv0, ru1, rv1, sc, sem0, sem1):
    wid = lax.axis_index("s") * _NC + lax.axis_index("c")
    base = wid * _EPW

    pltpu.sync_copy(u_hbm.at[pl.ds(base, _EPW)], iu)
    pltpu.sync_copy(v_hbm.at[pl.ds(base, _EPW)], iv)

    bufs = ((ru0, rv0, sem0), (ru1, rv1, sem1))

    def issue(blk, ru, rv, sem):
        off = blk * _B
        pltpu.async_copy(h_hbm.at[iu.at[pl.ds(off, _B)]], ru, sem)
        pltpu.async_copy(h_hbm.at[iv.at[pl.ds(off, _B)]], rv, sem)

    def drain(ru, rv, sem):
        pltpu.make_async_copy(h_hbm.at[iu.at[pl.ds(0, _B)]], ru, sem).wait()
        pltpu.make_async_copy(h_hbm.at[iv.at[pl.ds(0, _B)]], rv, sem).wait()

    issue(0, ru0, rv0, sem0)
    issue(1, ru1, rv1, sem1)

    def compute(blk, ru, rv):
        def g_body(g, carry):
            ev = lax.iota(jnp.int32, _LANES) + g * _LANES
            acc = jnp.zeros((_LANES,), jnp.float32)
            for d in range(_D):
                dv = jnp.full((_LANES,), d, jnp.int32)
                uu = plsc.load_gather(ru, [ev, dv])
                vv = plsc.load_gather(rv, [ev, dv])
                acc = acc + uu * vv
            sc[pl.ds(blk * _B + g * _LANES, _LANES)] = acc
            return carry
        lax.fori_loop(0, _B // _LANES, g_body, 0)

    def j_body(j, carry):
        for b in range(2):
            blk = j * 2 + b
            ru, rv, sem = bufs[b]
            drain(ru, rv, sem)
            compute(blk, ru, rv)
            nxt = blk + 2

            @pl.when(nxt < _NBLK)
            def _():
                issue(nxt, ru, rv, sem)
        return carry

    lax.fori_loop(0, _NBLK // 2, j_body, 0)

    pltpu.sync_copy(sc, out_hbm.at[pl.ds(base, _EPW)])


@jax.jit
def _sc_scores(h, u, v):
    mesh = plsc.VectorSubcoreMesh(core_axis_name="c", subcore_axis_name="s")
    return pl.kernel(
        _sc_scores_kernel,
        out_type=jax.ShapeDtypeStruct((_E,), jnp.float32),
        mesh=mesh,
        compiler_params=pltpu.CompilerParams(needs_layout_passes=False),
        scratch_types=[
            pltpu.VMEM((_EPW,), jnp.int32),
            pltpu.VMEM((_EPW,), jnp.int32),
            pltpu.VMEM((_B, _D), jnp.float32),
            pltpu.VMEM((_B, _D), jnp.float32),
            pltpu.VMEM((_B, _D), jnp.float32),
            pltpu.VMEM((_B, _D), jnp.float32),
            pltpu.VMEM((_EPW,), jnp.float32),
            pltpu.SemaphoreType.DMA,
            pltpu.SemaphoreType.DMA,
        ],
    )(h, u, v)


def _loss_body(s_ref, o_ref):
    s = s_ref[...]
    pos = s[: _N_POS // _D, :]
    neg = s[_N_POS // _D:, :]
    t = jnp.exp(-pos - _EPS)
    loss_edges = jnp.mean(jnp.log(1.0 + t))
    loss_nonedges = jnp.mean(neg)
    o_ref[...] = jnp.reshape(loss_edges + loss_nonedges, (1, 1))


@jax.jit
def _tc_loss(scores2d):
    out = pl.pallas_call(
        _loss_body,
        out_shape=jax.ShapeDtypeStruct((1, 1), jnp.float32),
    )(scores2d)
    return out[0, 0]


def kernel(block_outputs, pos_edge_index, neg_edge_index):
    u = jnp.concatenate([pos_edge_index[0], neg_edge_index[0]])
    v = jnp.concatenate([pos_edge_index[1], neg_edge_index[1]])
    scores = _sc_scores(block_outputs, u, v)
    return _tc_loss(scores.reshape(_E // _D, _D))
